# baseline (speedup 1.0000x reference)
import jax
import jax.numpy as jnp
from jax.experimental import pallas as pl

N_IN = 128
EMB_DIM = 512
ROWS = 256


def _embed_kernel(inp_ref, topo_ref, out_ref):
    c = pl.program_id(0)
    nper = topo_ref.shape[0] // ROWS
    g0 = (c % nper) * ROWS
    out_ref[:, 0, :N_IN] = inp_ref[0]
    out_ref[:, 0, N_IN:] = topo_ref[pl.ds(g0, ROWS), : EMB_DIM - N_IN]


def kernel(inputs, grid_positions, embedding, topographical_embedding, x_learn, y_learn):
    B, GRID, _ = inputs.shape
    nper = GRID // ROWS

    out = pl.pallas_call(
        _embed_kernel,
        grid=(B * nper,),
        in_specs=[
            pl.BlockSpec((1, ROWS, N_IN), lambda c: (c // nper, c % nper, 0)),
            pl.BlockSpec((GRID, EMB_DIM), lambda c: (0, 0)),
        ],
        out_specs=pl.BlockSpec((ROWS, 1, EMB_DIM), lambda c: (c, 0, 0)),
        out_shape=jax.ShapeDtypeStruct((B * GRID, 1, EMB_DIM), jnp.float32),
    )(inputs, topographical_embedding)
    return out


# speedup vs baseline: 1.0012x; 1.0012x over previous
import jax
import jax.numpy as jnp
from jax.experimental import pallas as pl

N_IN = 128
EMB_DIM = 512
ROWS = 256


def _embed_kernel(inp_ref, topo_ref, out_ref):
    out_ref[:, 0, :N_IN] = inp_ref[0]
    out_ref[:, 0, N_IN:] = topo_ref[:, : EMB_DIM - N_IN]


def kernel(inputs, grid_positions, embedding, topographical_embedding, x_learn, y_learn):
    B, GRID, _ = inputs.shape
    nper = GRID // ROWS

    out = pl.pallas_call(
        _embed_kernel,
        grid=(nper, B),
        in_specs=[
            pl.BlockSpec((1, ROWS, N_IN), lambda c, b: (b, c, 0)),
            pl.BlockSpec((ROWS, EMB_DIM), lambda c, b: (c, 0)),
        ],
        out_specs=pl.BlockSpec((ROWS, 1, EMB_DIM), lambda c, b: (b * nper + c, 0, 0)),
        out_shape=jax.ShapeDtypeStruct((B * GRID, 1, EMB_DIM), jnp.float32),
    )(inputs, topographical_embedding)
    return out


# speedup vs baseline: 2.5704x; 2.5673x over previous
import jax
import jax.numpy as jnp
from jax.experimental import pallas as pl

N_IN = 128
EMB_DIM = 512
BPG = 2


def _embed_kernel(inp_ref, topo_ref, out_ref):
    G = topo_ref.shape[0]
    for j in range(BPG):
        out_ref[pl.ds(j * G, G), 0, :N_IN] = inp_ref[j]
        out_ref[pl.ds(j * G, G), 0, N_IN:] = topo_ref[:, : EMB_DIM - N_IN]


def kernel(inputs, grid_positions, embedding, topographical_embedding, x_learn, y_learn):
    B, GRID, _ = inputs.shape

    out = pl.pallas_call(
        _embed_kernel,
        grid=(B // BPG,),
        in_specs=[
            pl.BlockSpec((BPG, GRID, N_IN), lambda c: (c, 0, 0)),
            pl.BlockSpec((GRID, EMB_DIM), lambda c: (0, 0)),
        ],
        out_specs=pl.BlockSpec((BPG * GRID, 1, EMB_DIM), lambda c: (c, 0, 0)),
        out_shape=jax.ShapeDtypeStruct((B * GRID, 1, EMB_DIM), jnp.float32),
    )(inputs, topographical_embedding)
    return out


# speedup vs baseline: 2.8525x; 1.1098x over previous
import jax
import jax.numpy as jnp
from jax.experimental import pallas as pl

N_IN = 128
EMB_DIM = 512
BPG = 4


def _embed_kernel(inp_ref, topo_ref, out_ref):
    G = topo_ref.shape[0]
    for j in range(BPG):
        out_ref[pl.ds(j * G, G), 0, :N_IN] = inp_ref[j]
        out_ref[pl.ds(j * G, G), 0, N_IN:] = topo_ref[:, : EMB_DIM - N_IN]


def kernel(inputs, grid_positions, embedding, topographical_embedding, x_learn, y_learn):
    B, GRID, _ = inputs.shape

    out = pl.pallas_call(
        _embed_kernel,
        grid=(B // BPG,),
        in_specs=[
            pl.BlockSpec((BPG, GRID, N_IN), lambda c: (c, 0, 0)),
            pl.BlockSpec((GRID, EMB_DIM), lambda c: (0, 0)),
        ],
        out_specs=pl.BlockSpec((BPG * GRID, 1, EMB_DIM), lambda c: (c, 0, 0)),
        out_shape=jax.ShapeDtypeStruct((B * GRID, 1, EMB_DIM), jnp.float32),
    )(inputs, topographical_embedding)
    return out
